# Initial kernel scaffold; baseline (speedup 1.0000x reference)
import jax
import jax.numpy as jnp
from jax.experimental import pallas as pl


def kernel(prediction, ix):
    raise NotImplementedError("write your pallas kernel here")


# speedup vs baseline: 6.4238x; 6.4238x over previous
import functools

import jax
import jax.numpy as jnp
from jax.experimental import pallas as pl

CONFIDENCE = 0.8
SMOOTHING = 1.0 - CONFIDENCE

ROW_TILE = 256
V_TILE = 6400


def _fill_kernel(ix_ref, out_ref, *, base, v_tile):
    j = pl.program_id(1)
    col0 = j * v_tile
    cols = jax.lax.broadcasted_iota(jnp.int32, out_ref.shape, 1) + col0
    ix = ix_ref[:, 0][:, None]
    out_ref[...] = jnp.where(cols == ix, CONFIDENCE, base).astype(out_ref.dtype)


def kernel(prediction, ix):
    B, S, V = prediction.shape
    R = B * S
    base = SMOOTHING / (V - 1)
    ix2 = ix.reshape(R, 1)

    out = pl.pallas_call(
        functools.partial(_fill_kernel, base=base, v_tile=V_TILE),
        grid=(R // ROW_TILE, V // V_TILE),
        in_specs=[pl.BlockSpec((ROW_TILE, 1), lambda i, j: (i, 0))],
        out_specs=pl.BlockSpec((ROW_TILE, V_TILE), lambda i, j: (i, j)),
        out_shape=jax.ShapeDtypeStruct((R, V), prediction.dtype),
    )(ix2)
    return out.reshape(B, S, V)
